# baseline (speedup 1.0000x reference)
import functools

import jax
import jax.numpy as jnp
from jax import lax
from jax.experimental import pallas as pl
from jax.experimental.pallas import tpu as pltpu
from jax.experimental.pallas import tpu_sc as plsc

_NC = 2
_NS = 16
_NW = _NC * _NS
_CHUNK = 8192
_NCHUNK = 12
_TAIL = 1696
_U = 8

_mesh = plsc.VectorSubcoreMesh(core_axis_name="c", subcore_axis_name="s",
                               num_cores=_NC, num_subcores=_NS)


def _sc_stream_body(x_hbm, m_out, s_out, buf0, buf1, tailbuf, stage_m,
                    stage_s, sh16, sem0, sem1, tsem, *, n_rows, v_total):
    wid = lax.axis_index("s") * _NC + lax.axis_index("c")
    rpw = n_rows // _NW
    base_row = wid * rpw
    iota16 = lax.broadcasted_iota(jnp.int32, (16,), 0)
    neg_inf = jnp.float32(-jnp.inf)

    bufs = (buf0, buf1)
    sems = (sem0, sem1)

    def start_chunk(row, c, b):
        pltpu.make_async_copy(
            x_hbm.at[row, pl.ds(c * _CHUNK, _CHUNK)],
            bufs[b], sems[b]).start()

    def wait_chunk(row, c, b):
        pltpu.make_async_copy(
            x_hbm.at[row, pl.ds(c * _CHUNK, _CHUNK)],
            bufs[b], sems[b]).wait()

    def max_scan(buf, nvec, m16):
        def step(g, m):
            for u in range(_U):
                m = jnp.maximum(m, buf[pl.ds((g * _U + u) * 16, 16)])
            return m
        return lax.fori_loop(0, nvec // _U, step, m16)

    def exp_scan(buf, nvec, m16, s16):
        def step(g, s):
            for u in range(_U):
                s = s + jnp.exp(buf[pl.ds((g * _U + u) * 16, 16)] - m16)
            return s
        return lax.fori_loop(0, nvec // _U, step, s16)

    def row_lse(row):
        pltpu.make_async_copy(
            x_hbm.at[row, pl.ds(_NCHUNK * _CHUNK, _TAIL)],
            tailbuf, tsem).start()
        start_chunk(row, jnp.int32(0), 0)
        start_chunk(row, jnp.int32(1), 1)

        def pair(g, carry):
            m16, s16 = carry
            for bb in range(2):
                c = g * 2 + bb
                wait_chunk(row, c, bb)
                start_chunk(row, jnp.minimum(c + 2, _NCHUNK - 2 + bb), bb)
                m_new = max_scan(bufs[bb], _CHUNK // 16, m16)
                s16 = s16 * jnp.exp(m16 - m_new)
                s16 = exp_scan(bufs[bb], _CHUNK // 16, m_new, s16)
                m16 = m_new
            return m16, s16

        m16 = jnp.full((16,), neg_inf, jnp.float32)
        s16 = jnp.zeros((16,), jnp.float32)
        m16, s16 = lax.fori_loop(0, _NCHUNK // 2, pair, (m16, s16))
        wait_chunk(row, jnp.int32(_NCHUNK - 2), 0)
        wait_chunk(row, jnp.int32(_NCHUNK - 1), 1)

        n_tv = _TAIL // 16
        n_tg = (n_tv // _U) * _U
        pltpu.make_async_copy(
            x_hbm.at[row, pl.ds(_NCHUNK * _CHUNK, _TAIL)],
            tailbuf, tsem).wait()
        m_new = max_scan(tailbuf, n_tg, m16)
        for u in range(n_tv - n_tg):
            m_new = jnp.maximum(m_new, tailbuf[pl.ds((n_tg + u) * 16, 16)])
        s16 = s16 * jnp.exp(m16 - m_new)
        s16 = exp_scan(tailbuf, n_tg, m_new, s16)
        for u in range(n_tv - n_tg):
            s16 = s16 + jnp.exp(tailbuf[pl.ds((n_tg + u) * 16, 16)] - m_new)
        m_rowv = m_new
        for sh in (1, 2, 4, 8):
            sh16[...] = m_rowv
            m_rowv = jnp.maximum(
                m_rowv, plsc.load_gather(sh16, [iota16 ^ sh]))
        s16 = s16 * jnp.exp(m_new - m_rowv)
        s_rowv = s16
        for sh in (1, 2, 4, 8):
            sh16[...] = s_rowv
            s_rowv = s_rowv + plsc.load_gather(sh16, [iota16 ^ sh])
        return m_rowv, s_rowv

    for half in range(rpw // 16):
        def row_body(rr, carry):
            rm, rs = carry
            m_rowv, s_rowv = row_lse(base_row + half * 16 + rr)
            sel = iota16 == rr
            rm = jnp.where(sel, m_rowv, rm)
            rs = jnp.where(sel, s_rowv, rs)
            return rm, rs

        rm = jnp.zeros((16,), jnp.float32)
        rs = jnp.full((16,), jnp.float32(1), jnp.float32)
        rm, rs = lax.fori_loop(0, 16, row_body, (rm, rs))
        stage_m[pl.ds(half * 16, 16)] = rm
        stage_s[pl.ds(half * 16, 16)] = rs

    pltpu.sync_copy(stage_m, m_out.at[pl.ds(base_row, rpw)])
    pltpu.sync_copy(stage_s, s_out.at[pl.ds(base_row, rpw)])


def _sc_pick_body(flat_hbm, t_hbm, p_out, tv, idx16, rows16, picked_v, sem,
                  *, n_rows, v_total):
    wid = lax.axis_index("s") * _NC + lax.axis_index("c")
    rpw = n_rows // _NW
    base = wid * rpw
    iota16 = lax.broadcasted_iota(jnp.int32, (16,), 0)

    pltpu.sync_copy(t_hbm.at[pl.ds(base, rpw)], tv)
    for u in range(rpw // 16):
        tvec = tv[pl.ds(16 * u, 16)]
        rowid = base + 16 * u + iota16
        flat = rowid * v_total + tvec
        idx16[...] = lax.shift_right_logical(flat, 7)
        lane = flat & 127
        pltpu.async_copy(flat_hbm.at[idx16], rows16, sem).wait()
        vals = plsc.load_gather(rows16, [iota16, lane])
        picked_v[pl.ds(16 * u, 16)] = vals
    pltpu.sync_copy(picked_v, p_out.at[pl.ds(base, rpw)])


def _finalize_body(m_ref, s_ref, p_ref, out_ref, *, k):
    loss = m_ref[...] + jnp.log(s_ref[...]) - p_ref[...]
    b = lax.bitcast_convert_type(loss, jnp.int32)
    key = b ^ (lax.shift_right_arithmetic(b, 31) & jnp.int32(0x7FFFFFFF))

    def cnt_ge(thresh):
        return jnp.sum((key >= thresh).astype(jnp.int32))

    base0 = jnp.where(cnt_ge(jnp.int32(0)) >= k, jnp.int32(0),
                      jnp.int32(-(2**31)))

    def body(i, base):
        cand = base | lax.shift_left(jnp.int32(1), 30 - i)
        return jnp.where(cnt_ge(cand) >= k, cand, base)

    big_t = lax.fori_loop(0, 31, body, base0)
    tb = big_t ^ (lax.shift_right_arithmetic(big_t, 31) & jnp.int32(0x7FFFFFFF))
    tval = lax.bitcast_convert_type(tb, jnp.float32)
    gt = loss > tval
    cnt_gt = jnp.sum(gt.astype(jnp.float32))
    sum_gt = jnp.sum(jnp.where(gt, loss, 0.0))
    res = (sum_gt + (jnp.float32(k) - cnt_gt) * tval) / jnp.float32(k)
    out_ref[...] = jnp.full((1, 1), res, jnp.float32)


@jax.jit
def kernel(inputs, targets):
    n, v = inputs.shape
    k = int(0.25 * n)
    rpw = n // _NW

    sc_stream = functools.partial(
        pl.kernel,
        out_type=(jax.ShapeDtypeStruct((n,), jnp.float32),
                  jax.ShapeDtypeStruct((n,), jnp.float32)),
        mesh=_mesh,
        compiler_params=pltpu.CompilerParams(needs_layout_passes=False),
        scratch_types=[
            pltpu.VMEM((_CHUNK,), jnp.float32),
            pltpu.VMEM((_CHUNK,), jnp.float32),
            pltpu.VMEM((_TAIL,), jnp.float32),
            pltpu.VMEM((rpw,), jnp.float32),
            pltpu.VMEM((rpw,), jnp.float32),
            pltpu.VMEM((16,), jnp.float32),
            pltpu.SemaphoreType.DMA,
            pltpu.SemaphoreType.DMA,
            pltpu.SemaphoreType.DMA,
        ],
    )(functools.partial(_sc_stream_body, n_rows=n, v_total=v))
    m, s = sc_stream(inputs)

    flat = inputs.reshape(n * v // 128, 128)
    sc_pick = functools.partial(
        pl.kernel,
        out_type=jax.ShapeDtypeStruct((n,), jnp.float32),
        mesh=_mesh,
        compiler_params=pltpu.CompilerParams(needs_layout_passes=False),
        scratch_types=[
            pltpu.VMEM((rpw,), jnp.int32),
            pltpu.VMEM((16,), jnp.int32),
            pltpu.VMEM((16, 128), jnp.float32),
            pltpu.VMEM((rpw,), jnp.float32),
            pltpu.SemaphoreType.DMA,
        ],
    )(functools.partial(_sc_pick_body, n_rows=n, v_total=v))
    picked = sc_pick(flat, targets.astype(jnp.int32))

    out = pl.pallas_call(
        functools.partial(_finalize_body, k=k),
        out_shape=jax.ShapeDtypeStruct((1, 1), jnp.float32),
    )(m.reshape(8, n // 8), s.reshape(8, n // 8), picked.reshape(8, n // 8))
    return out[0, 0]
